# baseline (speedup 1.0000x reference)
import jax
import jax.numpy as jnp
from jax import lax
from jax.experimental import pallas as pl
from jax.experimental.pallas import tpu as pltpu
from jax.experimental.pallas import tpu_sc as plsc

_NC = 2
_NS = 16
_NW = _NC * _NS
_L = 16

_N = 10000
_E = 320000
_D = 128
_H = _N // 2

_NH_PAD = 5120
_LTRASH = _H
_EPT = _E // _NW

_CWA = 128
_CREG = 81
_RSZ = _CREG * _CWA

_RPT = _NH_PAD // _NS
_FR = 40


def _deg_body(fei_hbm, clus_hbm, hist_out, csrc_out, cdst_out, cnt_out,
              src_v, dst_v, clus_v, hist_v, csa_v, cda_v, csb_v, cdb_v, cnt_v):
    c = lax.axis_index("c")
    s = lax.axis_index("s")
    wid = s * _NC + c
    pltpu.sync_copy(clus_hbm, clus_v)
    pltpu.sync_copy(fei_hbm.at[0, pl.ds(wid * _EPT, _EPT)], src_v)
    pltpu.sync_copy(fei_hbm.at[1, pl.ds(wid * _EPT, _EPT)], dst_v)

    zeros16 = jnp.zeros((_L,), jnp.float32)

    @pl.loop(0, _N // _L)
    def _zero(i):
        hist_v[pl.ds(i * _L, _L)] = zeros16

    zeros16i = jnp.zeros((_L,), jnp.int32)
    ltrash16 = jnp.full((_L,), _LTRASH, jnp.int32)

    @pl.loop(0, _RSZ // _L)
    def _pre(i):
        sl = pl.ds(i * _L, _L)
        csa_v[sl] = zeros16i
        cda_v[sl] = ltrash16
        csb_v[sl] = zeros16i
        cdb_v[sl] = ltrash16

    ones16 = jnp.ones((_L,), jnp.float32)
    h16 = jnp.full((_L,), _H, jnp.int32)

    @pl.loop(0, _EPT // _L, init_carry=(jnp.int32(0), jnp.int32(0)))
    def _group(g, carry):
        offa, offb = carry
        sl = pl.ds(g * _L, _L)
        sidx = src_v[sl]
        didx = dst_v[sl]
        cs = plsc.load_gather(clus_v, [sidx])
        cd = plsc.load_gather(clus_v, [didx])
        m = cs == cd
        plsc.addupdate_scatter(hist_v, [didx], ones16, mask=m)
        lo = didx < h16
        ma = m & lo
        mb = m & (~lo)
        dloc = jnp.where(lo, didx, didx - h16)
        plsc.store_compressed(csa_v.at[pl.ds(offa, _L)], sidx, mask=ma)
        plsc.store_compressed(cda_v.at[pl.ds(offa, _L)], dloc, mask=ma)
        plsc.store_compressed(csb_v.at[pl.ds(offb, _L)], sidx, mask=mb)
        plsc.store_compressed(cdb_v.at[pl.ds(offb, _L)], dloc, mask=mb)
        offa = offa + plsc.all_reduce_population_count(ma)[0]
        offb = offb + plsc.all_reduce_population_count(mb)[0]
        return offa, offb

    offa, offb = _group
    ncha = (offa + _CWA - 1) // _CWA
    nchb = (offb + _CWA - 1) // _CWA
    lane = lax.iota(jnp.int32, _L)
    cnt_v[...] = jnp.where(lane == 0, jnp.full((_L,), ncha, jnp.int32),
                           jnp.full((_L,), nchb, jnp.int32))

    pltpu.sync_copy(hist_v, hist_out.at[wid])
    pltpu.sync_copy(csa_v.at[pl.ds(0, _RSZ)], csrc_out.at[0, wid])
    pltpu.sync_copy(cda_v.at[pl.ds(0, _RSZ)], cdst_out.at[0, wid])
    pltpu.sync_copy(csb_v.at[pl.ds(0, _RSZ)], csrc_out.at[1, wid])
    pltpu.sync_copy(cdb_v.at[pl.ds(0, _RSZ)], cdst_out.at[1, wid])
    pltpu.sync_copy(cnt_v, cnt_out.at[wid])


def _agg_body(y_hbm, csrc_hbm, cdst_hbm, cnt_hbm, dinv_hbm, b_hbm, out_hbm,
              src_v, dst_v, cnt_v, rows0, rows1, dinv_v, bh_v, agg_sh,
              sem0, sem1):
    c = lax.axis_index("c")
    s = lax.axis_index("s")

    @pl.when(s < _NS - 1)
    def _init_full():
        pltpu.sync_copy(y_hbm.at[pl.ds(c * _H + s * _RPT, _RPT)],
                        agg_sh.at[pl.ds(s * _RPT, _RPT)])

    @pl.when(s == _NS - 1)
    def _init_tail():
        pltpu.sync_copy(
            y_hbm.at[pl.ds(c * _H + (_NS - 1) * _RPT, _H - (_NS - 1) * _RPT)],
            agg_sh.at[pl.ds((_NS - 1) * _RPT, _H - (_NS - 1) * _RPT)])

    pltpu.sync_copy(cnt_hbm.at[s], cnt_v)
    pltpu.sync_copy(csrc_hbm.at[c, 2 * s], src_v.at[0])
    pltpu.sync_copy(csrc_hbm.at[c, 2 * s + 1], src_v.at[1])
    pltpu.sync_copy(cdst_hbm.at[c, 2 * s], dst_v.at[0])
    pltpu.sync_copy(cdst_hbm.at[c, 2 * s + 1], dst_v.at[1])
    r0 = cnt_v[0]
    r1 = cnt_v[1]
    n0 = jnp.where(c == 0, r0[0], r0[1])
    n1 = jnp.where(c == 0, r1[0], r1[1])
    tot = n0 + n1
    npair = (jnp.maximum(tot, 1) + 1) // 2
    last = 2 * npair

    def chref(arr, j):
        in0 = j < n0
        inr = j < tot
        r_sel = jnp.where(in0 | (~inr), 0, 1)
        ch = jnp.where(in0, j, jnp.where(inr, j - n0, _CREG - 1))
        return arr.at[r_sel, ch]

    plsc.subcore_barrier()
    pltpu.async_copy(y_hbm.at[chref(src_v, 0)], rows0, sem0)

    @pl.loop(0, npair)
    def _pipe(i):
        j0 = 2 * i
        j1 = j0 + 1
        pltpu.async_copy(y_hbm.at[chref(src_v, j1)], rows1, sem1)
        pltpu.make_async_copy(y_hbm.at[chref(src_v, j0)], rows0, sem0).wait()
        pltpu.sync_copy(rows0, agg_sh.at[chref(dst_v, j0)], add=True)

        @pl.when(j1 + 1 < last)
        def _start_next():
            pltpu.async_copy(y_hbm.at[chref(src_v, j1 + 1)], rows0, sem0)

        pltpu.make_async_copy(y_hbm.at[chref(src_v, j1)], rows1, sem1).wait()
        pltpu.sync_copy(rows1, agg_sh.at[chref(dst_v, j1)], add=True)

    plsc.subcore_barrier()
    pltpu.sync_copy(b_hbm, bh_v)
    nsub = jnp.where(s == _NS - 1, (_H - (_NS - 1) * _RPT) // _FR, _RPT // _FR)

    @pl.loop(0, nsub)
    def _fin(i):
        loc = s * _RPT + i * _FR
        base = c * _H + loc
        pltpu.sync_copy(agg_sh.at[pl.ds(loc, _FR)], rows0.at[pl.ds(0, _FR)])
        pltpu.sync_copy(dinv_hbm.at[pl.ds(base, _FR)], dinv_v.at[pl.ds(0, _FR)])
        for g in range((_FR + _L - 1) // _L):
            dvec = dinv_v[pl.ds(g * _L, _L)]
            for k in range(min(_L, _FR - g * _L)):
                row = g * _L + k
                dscal = dvec[k]
                for q in range(_D // _L):
                    sl = pl.ds(q * _L, _L)
                    rows0[row, sl] = rows0[row, sl] * dscal + bh_v[sl]
        pltpu.sync_copy(rows0.at[pl.ds(0, _FR)], out_hbm.at[pl.ds(base, _FR)])


def _dy_body(hist_ref, x_ref, w_ref, y_ref, dinv_ref):
    deg = jnp.sum(hist_ref[...], axis=0) + 1.0
    dinv = lax.rsqrt(deg)
    dinv_ref[...] = dinv
    xw = jnp.dot(x_ref[...], w_ref[...], preferred_element_type=jnp.float32)
    y_ref[...] = xw * dinv[:, None]


def _sc_mesh():
    return plsc.VectorSubcoreMesh(core_axis_name="c", subcore_axis_name="s")


def _deg_call(fei, clus):
    f = pl.kernel(
        _deg_body,
        out_type=(
            jax.ShapeDtypeStruct((_NW, _N), jnp.float32),
            jax.ShapeDtypeStruct((_NC, _NW, _RSZ), jnp.int32),
            jax.ShapeDtypeStruct((_NC, _NW, _RSZ), jnp.int32),
            jax.ShapeDtypeStruct((_NW, _L), jnp.int32),
        ),
        mesh=_sc_mesh(),
        scratch_types=[
            pltpu.VMEM((_EPT,), jnp.int32),
            pltpu.VMEM((_EPT,), jnp.int32),
            pltpu.VMEM((_N,), jnp.int32),
            pltpu.VMEM((_N,), jnp.float32),
            pltpu.VMEM((_RSZ + _L,), jnp.int32),
            pltpu.VMEM((_RSZ + _L,), jnp.int32),
            pltpu.VMEM((_RSZ + _L,), jnp.int32),
            pltpu.VMEM((_RSZ + _L,), jnp.int32),
            pltpu.VMEM((_L,), jnp.int32),
        ],
        compiler_params=pltpu.CompilerParams(
            needs_layout_passes=False, use_tc_tiling_on_sc=False),
    )
    return f(fei, clus)


def _agg_call(y, csrc, cdst, cnt, dinv, b):
    f = pl.kernel(
        _agg_body,
        out_type=jax.ShapeDtypeStruct((_N, _D), jnp.float32),
        mesh=_sc_mesh(),
        scratch_types=[
            pltpu.VMEM((2, _CREG, _CWA), jnp.int32),
            pltpu.VMEM((2, _CREG, _CWA), jnp.int32),
            pltpu.VMEM((2, _L), jnp.int32),
            pltpu.VMEM((_CWA, _D), jnp.float32),
            pltpu.VMEM((_CWA, _D), jnp.float32),
            pltpu.VMEM((3 * _L,), jnp.float32),
            pltpu.VMEM((_D,), jnp.float32),
            pltpu.VMEM_SHARED((_NH_PAD, _D), jnp.float32),
            pltpu.SemaphoreType.DMA,
            pltpu.SemaphoreType.DMA,
        ],
        compiler_params=pltpu.CompilerParams(
            needs_layout_passes=False, use_tc_tiling_on_sc=False),
    )
    return f(y, csrc, cdst, cnt, dinv, b)


def _dy_call(hist, x, w):
    return pl.pallas_call(
        _dy_body,
        out_shape=(
            jax.ShapeDtypeStruct((_N, _D), jnp.float32),
            jax.ShapeDtypeStruct((_N,), jnp.float32),
        ),
    )(hist, x, w)


def kernel(X, W, b, cluster_assignment, full_edge_index):
    hist, csrc, cdst, cnt = _deg_call(full_edge_index, cluster_assignment)
    y, dinv = _dy_call(hist, X, W)
    return _agg_call(y,
                     csrc.reshape(_NC, _NW, _CREG, _CWA),
                     cdst.reshape(_NC, _NW, _CREG, _CWA),
                     cnt.reshape(_NS, 2, _L),
                     dinv, b)


# speedup vs baseline: 1.7224x; 1.7224x over previous
import jax
import jax.numpy as jnp
from jax import lax
from jax.experimental import pallas as pl
from jax.experimental.pallas import tpu as pltpu
from jax.experimental.pallas import tpu_sc as plsc

_NC = 2
_NS = 16
_NW = _NC * _NS
_L = 16

_N = 10000
_E = 320000
_D = 128
_DH = _D // 2

_N_PAD = 10240
_TRASH = _N
_EPT = _E // _NW

_CWA = 128
_CREG = 81
_RSZ = _CREG * _CWA

_RPT = _N_PAD // _NS
_FR = 80
_BR = 400


def _deg_body(fei_hbm, clus_hbm, hist_out, csrc_out, cdst_out, cnt_out,
              src_v, dst_v, clus_v, hist_v, csrc_v, cdst_v, cnt_v):
    c = lax.axis_index("c")
    s = lax.axis_index("s")
    wid = s * _NC + c
    pltpu.sync_copy(clus_hbm, clus_v)
    pltpu.sync_copy(fei_hbm.at[0, pl.ds(wid * _EPT, _EPT)], src_v)
    pltpu.sync_copy(fei_hbm.at[1, pl.ds(wid * _EPT, _EPT)], dst_v)

    zeros16 = jnp.zeros((_L,), jnp.float32)

    @pl.loop(0, _N // _L)
    def _zero(i):
        hist_v[pl.ds(i * _L, _L)] = zeros16

    zeros16i = jnp.zeros((_L,), jnp.int32)
    trash16 = jnp.full((_L,), _TRASH, jnp.int32)

    @pl.loop(0, _RSZ // _L)
    def _pre(i):
        csrc_v[pl.ds(i * _L, _L)] = zeros16i
        cdst_v[pl.ds(i * _L, _L)] = trash16

    ones16 = jnp.ones((_L,), jnp.float32)

    @pl.loop(0, _EPT // _L, init_carry=jnp.int32(0), unroll=4)
    def _group(g, off):
        sl = pl.ds(g * _L, _L)
        sidx = src_v[sl]
        didx = dst_v[sl]
        cs = plsc.load_gather(clus_v, [sidx])
        cd = plsc.load_gather(clus_v, [didx])
        m = cs == cd
        plsc.addupdate_scatter(hist_v, [didx], ones16, mask=m)
        plsc.store_compressed(csrc_v.at[pl.ds(off, _L)], sidx, mask=m)
        plsc.store_compressed(cdst_v.at[pl.ds(off, _L)], didx, mask=m)
        return off + plsc.all_reduce_population_count(m)[0]

    off = _group
    nch = (off + _CWA - 1) // _CWA
    cnt_v[...] = jnp.full((_L,), nch, jnp.int32)

    pltpu.sync_copy(hist_v, hist_out.at[wid])
    pltpu.sync_copy(csrc_v.at[pl.ds(0, _RSZ)], csrc_out.at[wid])
    pltpu.sync_copy(cdst_v.at[pl.ds(0, _RSZ)], cdst_out.at[wid])
    pltpu.sync_copy(cnt_v, cnt_out.at[wid])


def _agg_body(y_hbm, csrc_hbm, cdst_hbm, cnt_hbm, dinv_hbm, b_hbm, out_hbm,
              src_v, dst_v, cnt_v, rows0, rows1, dinv_v, bh_v, agg_sh,
              sem0, sem1, sem2):

    c = lax.axis_index("c")
    s = lax.axis_index("s")

    zeros16 = jnp.zeros((_L,), jnp.float32)

    @pl.loop(0, _CWA)
    def _zbuf(i):
        for k in range(_DH // _L):
            rows0[i, pl.ds(k * _L, _L)] = zeros16

    @pl.loop(0, _RPT // _CWA)
    def _zstripe(i):
        pltpu.sync_copy(rows0, agg_sh.at[pl.ds(s * _RPT + i * _CWA, _CWA)])

    ytab = y_hbm.at[c]
    pltpu.sync_copy(cnt_hbm.at[s], cnt_v)
    pltpu.sync_copy(csrc_hbm.at[2 * s], src_v.at[0])
    pltpu.sync_copy(csrc_hbm.at[2 * s + 1], src_v.at[1])
    pltpu.sync_copy(cdst_hbm.at[2 * s], dst_v.at[0])
    pltpu.sync_copy(cdst_hbm.at[2 * s + 1], dst_v.at[1])
    n0 = cnt_v[0][0]
    n1 = cnt_v[1][0]
    tot = n0 + n1
    npair = (jnp.maximum(tot, 1) + 1) // 2
    last = 2 * npair

    def chref(arr, j):
        in0 = j < n0
        inr = j < tot
        r_sel = jnp.where(in0 | (~inr), 0, 1)
        ch = jnp.where(in0, j, jnp.where(inr, j - n0, _CREG - 1))
        return arr.at[r_sel, ch]

    plsc.subcore_barrier()
    pltpu.async_copy(ytab.at[chref(src_v, 0)], rows0, sem0)

    @pl.loop(0, npair)
    def _pipe(i):
        j0 = 2 * i
        j1 = j0 + 1
        pltpu.async_copy(ytab.at[chref(src_v, j1)], rows1, sem1)
        pltpu.make_async_copy(ytab.at[chref(src_v, j0)], rows0, sem0).wait()
        pltpu.sync_copy(rows0, agg_sh.at[chref(dst_v, j0)], add=True)

        @pl.when(j1 + 1 < last)
        def _start_next():
            pltpu.async_copy(ytab.at[chref(src_v, j1 + 1)], rows0, sem0)

        pltpu.make_async_copy(ytab.at[chref(src_v, j1)], rows1, sem1).wait()
        pltpu.sync_copy(rows1, agg_sh.at[chref(dst_v, j1)], add=True)

    plsc.subcore_barrier()
    pltpu.sync_copy(b_hbm.at[pl.ds(c * _DH, _DH)], bh_v)
    nsub = jnp.where(s == _NS - 1, (_N - (_NS - 1) * _RPT) // _FR, _RPT // _FR)

    @pl.loop(0, nsub)
    def _fin(i):
        base = s * _RPT + i * _FR
        ca = pltpu.async_copy(agg_sh.at[pl.ds(base, _FR)], rows0.at[pl.ds(0, _FR)], sem0)
        cy = pltpu.async_copy(y_hbm.at[c, pl.ds(base, _FR)], rows1.at[pl.ds(0, _FR)], sem1)
        cd = pltpu.async_copy(dinv_hbm.at[pl.ds(base, _FR)], dinv_v, sem2)
        ca.wait()
        cy.wait()
        cd.wait()
        for g in range(_FR // _L):
            dvec = dinv_v[pl.ds(g * _L, _L)]
            for k in range(_L):
                row = g * _L + k
                dscal = dvec[k]
                for q in range(_DH // _L):
                    sl = pl.ds(q * _L, _L)
                    rows0[row, sl] = (rows0[row, sl] + rows1[row, sl]) * dscal + bh_v[sl]
        pltpu.sync_copy(rows0.at[pl.ds(0, _FR)],
                        out_hbm.at[pl.ds(base, _FR), pl.ds(c * _DH, _DH)])


def _dy_body(hist_ref, x_ref, w_ref, y_ref, dinv_ref):
    deg = jnp.sum(hist_ref[...], axis=0) + 1.0
    dinv = lax.rsqrt(deg)
    dinv_ref[...] = dinv
    xw = jnp.dot(x_ref[...], w_ref[...], preferred_element_type=jnp.float32)
    y = xw * dinv[:, None]
    y_ref[0] = y[:, :_DH]
    y_ref[1] = y[:, _DH:]


def _sc_mesh():
    return plsc.VectorSubcoreMesh(core_axis_name="c", subcore_axis_name="s")


def _deg_call(fei, clus):
    f = pl.kernel(
        _deg_body,
        out_type=(
            jax.ShapeDtypeStruct((_NW, _N), jnp.float32),
            jax.ShapeDtypeStruct((_NW, _RSZ), jnp.int32),
            jax.ShapeDtypeStruct((_NW, _RSZ), jnp.int32),
            jax.ShapeDtypeStruct((_NW, _L), jnp.int32),
        ),
        mesh=_sc_mesh(),
        scratch_types=[
            pltpu.VMEM((_EPT,), jnp.int32),
            pltpu.VMEM((_EPT,), jnp.int32),
            pltpu.VMEM((_N,), jnp.int32),
            pltpu.VMEM((_N,), jnp.float32),
            pltpu.VMEM((_RSZ + _L,), jnp.int32),
            pltpu.VMEM((_RSZ + _L,), jnp.int32),
            pltpu.VMEM((_L,), jnp.int32),
        ],
        compiler_params=pltpu.CompilerParams(
            needs_layout_passes=False, use_tc_tiling_on_sc=False),
    )
    return f(fei, clus)


def _agg_call(y2, csrc_a, cdst_a, cnt, dinv, b):
    f = pl.kernel(
        _agg_body,
        out_type=jax.ShapeDtypeStruct((_N, _D), jnp.float32),
        mesh=_sc_mesh(),
        scratch_types=[
            pltpu.VMEM((2, _CREG, _CWA), jnp.int32),
            pltpu.VMEM((2, _CREG, _CWA), jnp.int32),
            pltpu.VMEM((2, _L), jnp.int32),
            pltpu.VMEM((_CWA, _DH), jnp.float32),
            pltpu.VMEM((_CWA, _DH), jnp.float32),
            pltpu.VMEM((_FR,), jnp.float32),
            pltpu.VMEM((_DH,), jnp.float32),
            pltpu.VMEM_SHARED((_N_PAD, _DH), jnp.float32),
            pltpu.SemaphoreType.DMA,
            pltpu.SemaphoreType.DMA,
            pltpu.SemaphoreType.DMA,
        ],
        compiler_params=pltpu.CompilerParams(
            needs_layout_passes=False, use_tc_tiling_on_sc=False),
    )
    return f(y2, csrc_a, cdst_a, cnt, dinv, b)


def _dy_call(hist, x, w):
    return pl.pallas_call(
        _dy_body,
        out_shape=(
            jax.ShapeDtypeStruct((_NC, _N, _DH), jnp.float32),
            jax.ShapeDtypeStruct((_N,), jnp.float32),
        ),
    )(hist, x, w)


def kernel(X, W, b, cluster_assignment, full_edge_index):
    n, d = X.shape
    hist, csrc, cdst, cnt = _deg_call(full_edge_index, cluster_assignment)
    y2, dinv = _dy_call(hist, X, W)
    return _agg_call(y2,
                     csrc.reshape(_NW, _CREG, _CWA),
                     cdst.reshape(_NW, _CREG, _CWA),
                     cnt.reshape(_NS, 2, _L),
                     dinv, b)


# speedup vs baseline: 1.7468x; 1.0142x over previous
import jax
import jax.numpy as jnp
from jax import lax
from jax.experimental import pallas as pl
from jax.experimental.pallas import tpu as pltpu
from jax.experimental.pallas import tpu_sc as plsc

_NC = 2
_NS = 16
_NW = _NC * _NS
_L = 16

_N = 10000
_E = 320000
_D = 128
_DH = _D // 2

_N_PAD = 10240
_TRASH = _N
_EPT = _E // _NW

_CWA = 128
_CREG = 81
_RSZ = _CREG * _CWA

_RPT = _N_PAD // _NS
_FR = 80
_BR = 400


def _deg_body(fei_hbm, clus_hbm, hist_out, csrc_out, cdst_out, cnt_out,
              src_v, dst_v, clus_v, hist_v, csrc_v, cdst_v, cnt_v):
    c = lax.axis_index("c")
    s = lax.axis_index("s")
    wid = s * _NC + c
    pltpu.sync_copy(clus_hbm, clus_v)
    pltpu.sync_copy(fei_hbm.at[0, pl.ds(wid * _EPT, _EPT)], src_v)
    pltpu.sync_copy(fei_hbm.at[1, pl.ds(wid * _EPT, _EPT)], dst_v)

    zeros16 = jnp.zeros((_L,), jnp.float32)

    @pl.loop(0, _N // _L)
    def _zero(i):
        hist_v[pl.ds(i * _L, _L)] = zeros16

    zeros16i = jnp.zeros((_L,), jnp.int32)
    trash16 = jnp.full((_L,), _TRASH, jnp.int32)

    @pl.loop(0, _RSZ // _L)
    def _pre(i):
        csrc_v[pl.ds(i * _L, _L)] = zeros16i
        cdst_v[pl.ds(i * _L, _L)] = trash16

    ones16 = jnp.ones((_L,), jnp.float32)

    @pl.loop(0, _EPT // _L, init_carry=jnp.int32(0), unroll=4)
    def _group(g, off):
        sl = pl.ds(g * _L, _L)
        sidx = src_v[sl]
        didx = dst_v[sl]
        cs = plsc.load_gather(clus_v, [sidx])
        cd = plsc.load_gather(clus_v, [didx])
        m = cs == cd
        plsc.addupdate_scatter(hist_v, [didx], ones16, mask=m)
        plsc.store_compressed(csrc_v.at[pl.ds(off, _L)], sidx, mask=m)
        plsc.store_compressed(cdst_v.at[pl.ds(off, _L)], didx, mask=m)
        return off + plsc.all_reduce_population_count(m)[0]

    off = _group
    nch = (off + _CWA - 1) // _CWA
    cnt_v[...] = jnp.full((_L,), nch, jnp.int32)

    pltpu.sync_copy(hist_v, hist_out.at[wid])
    pltpu.sync_copy(csrc_v.at[pl.ds(0, _RSZ)], csrc_out.at[wid])
    pltpu.sync_copy(cdst_v.at[pl.ds(0, _RSZ)], cdst_out.at[wid])
    pltpu.sync_copy(cnt_v, cnt_out.at[wid])


def _agg_body(y_hbm, csrc_hbm, cdst_hbm, cnt_hbm, dinv_hbm, b_hbm, out_hbm,
              src_v, dst_v, cnt_v, rows0, rows1, dinv_v, bh_v, agg_sh,
              sem0, sem1, sem2, sem3):

    c = lax.axis_index("c")
    s = lax.axis_index("s")

    zeros16 = jnp.zeros((_L,), jnp.float32)

    @pl.loop(0, _CWA)
    def _zbuf(i):
        for k in range(_DH // _L):
            rows0[i, pl.ds(k * _L, _L)] = zeros16

    @pl.loop(0, _RPT // _CWA)
    def _zstripe(i):
        pltpu.sync_copy(rows0, agg_sh.at[pl.ds(s * _RPT + i * _CWA, _CWA)])

    ytab = y_hbm.at[c]
    l0 = pltpu.async_copy(csrc_hbm.at[2 * s], src_v.at[0], sem0)
    l1 = pltpu.async_copy(csrc_hbm.at[2 * s + 1], src_v.at[1], sem1)
    l2 = pltpu.async_copy(cdst_hbm.at[2 * s], dst_v.at[0], sem2)
    l3 = pltpu.async_copy(cdst_hbm.at[2 * s + 1], dst_v.at[1], sem3)
    pltpu.sync_copy(cnt_hbm.at[s], cnt_v)
    l0.wait()
    l1.wait()
    l2.wait()
    l3.wait()
    n0 = cnt_v[0][0]
    n1 = cnt_v[1][0]
    tot = n0 + n1
    npair = (jnp.maximum(tot, 1) + 1) // 2
    last = 2 * npair

    def chref(arr, j):
        in0 = j < n0
        inr = j < tot
        r_sel = jnp.where(in0 | (~inr), 0, 1)
        ch = jnp.where(in0, j, jnp.where(inr, j - n0, _CREG - 1))
        return arr.at[r_sel, ch]

    plsc.subcore_barrier()
    pltpu.async_copy(ytab.at[chref(src_v, 0)], rows0, sem0)

    @pl.loop(0, npair)
    def _pipe(i):
        j0 = 2 * i
        j1 = j0 + 1
        pltpu.async_copy(ytab.at[chref(src_v, j1)], rows1, sem1)
        pltpu.make_async_copy(ytab.at[chref(src_v, j0)], rows0, sem0).wait()
        pltpu.sync_copy(rows0, agg_sh.at[chref(dst_v, j0)], add=True)

        @pl.when(j1 + 1 < last)
        def _start_next():
            pltpu.async_copy(ytab.at[chref(src_v, j1 + 1)], rows0, sem0)

        pltpu.make_async_copy(ytab.at[chref(src_v, j1)], rows1, sem1).wait()
        pltpu.sync_copy(rows1, agg_sh.at[chref(dst_v, j1)], add=True)

    plsc.subcore_barrier()
    pltpu.sync_copy(b_hbm.at[pl.ds(c * _DH, _DH)], bh_v)
    nsub = jnp.where(s == _NS - 1, (_N - (_NS - 1) * _RPT) // _FR, _RPT // _FR)

    @pl.loop(0, nsub)
    def _fin(i):
        base = s * _RPT + i * _FR
        ca = pltpu.async_copy(agg_sh.at[pl.ds(base, _FR)], rows0.at[pl.ds(0, _FR)], sem0)
        cy = pltpu.async_copy(y_hbm.at[c, pl.ds(base, _FR)], rows1.at[pl.ds(0, _FR)], sem1)
        cd = pltpu.async_copy(dinv_hbm.at[pl.ds(base, _FR)], dinv_v, sem2)
        ca.wait()
        cy.wait()
        cd.wait()
        for g in range(_FR // _L):
            dvec = dinv_v[pl.ds(g * _L, _L)]
            for k in range(_L):
                row = g * _L + k
                dscal = dvec[k]
                for q in range(_DH // _L):
                    sl = pl.ds(q * _L, _L)
                    rows0[row, sl] = (rows0[row, sl] + rows1[row, sl]) * dscal + bh_v[sl]
        pltpu.sync_copy(rows0.at[pl.ds(0, _FR)],
                        out_hbm.at[pl.ds(base, _FR), pl.ds(c * _DH, _DH)])


def _dy_body(hist_ref, x_ref, w_ref, y_ref, dinv_ref):
    deg = jnp.sum(hist_ref[...], axis=0) + 1.0
    dinv = lax.rsqrt(deg)
    dinv_ref[...] = dinv
    xw = jnp.dot(x_ref[...], w_ref[...], preferred_element_type=jnp.float32)
    y = xw * dinv[:, None]
    y_ref[0] = y[:, :_DH]
    y_ref[1] = y[:, _DH:]


def _sc_mesh():
    return plsc.VectorSubcoreMesh(core_axis_name="c", subcore_axis_name="s")


def _deg_call(fei, clus):
    f = pl.kernel(
        _deg_body,
        out_type=(
            jax.ShapeDtypeStruct((_NW, _N), jnp.float32),
            jax.ShapeDtypeStruct((_NW, _RSZ), jnp.int32),
            jax.ShapeDtypeStruct((_NW, _RSZ), jnp.int32),
            jax.ShapeDtypeStruct((_NW, _L), jnp.int32),
        ),
        mesh=_sc_mesh(),
        scratch_types=[
            pltpu.VMEM((_EPT,), jnp.int32),
            pltpu.VMEM((_EPT,), jnp.int32),
            pltpu.VMEM((_N,), jnp.int32),
            pltpu.VMEM((_N,), jnp.float32),
            pltpu.VMEM((_RSZ + _L,), jnp.int32),
            pltpu.VMEM((_RSZ + _L,), jnp.int32),
            pltpu.VMEM((_L,), jnp.int32),
        ],
        compiler_params=pltpu.CompilerParams(
            needs_layout_passes=False, use_tc_tiling_on_sc=False),
    )
    return f(fei, clus)


def _agg_call(y2, csrc_a, cdst_a, cnt, dinv, b):
    f = pl.kernel(
        _agg_body,
        out_type=jax.ShapeDtypeStruct((_N, _D), jnp.float32),
        mesh=_sc_mesh(),
        scratch_types=[
            pltpu.VMEM((2, _CREG, _CWA), jnp.int32),
            pltpu.VMEM((2, _CREG, _CWA), jnp.int32),
            pltpu.VMEM((2, _L), jnp.int32),
            pltpu.VMEM((_CWA, _DH), jnp.float32),
            pltpu.VMEM((_CWA, _DH), jnp.float32),
            pltpu.VMEM((_FR,), jnp.float32),
            pltpu.VMEM((_DH,), jnp.float32),
            pltpu.VMEM_SHARED((_N_PAD, _DH), jnp.float32),
            pltpu.SemaphoreType.DMA,
            pltpu.SemaphoreType.DMA,
            pltpu.SemaphoreType.DMA,
            pltpu.SemaphoreType.DMA,
        ],
        compiler_params=pltpu.CompilerParams(
            needs_layout_passes=False, use_tc_tiling_on_sc=False),
    )
    return f(y2, csrc_a, cdst_a, cnt, dinv, b)


def _dy_call(hist, x, w):
    return pl.pallas_call(
        _dy_body,
        out_shape=(
            jax.ShapeDtypeStruct((_NC, _N, _DH), jnp.float32),
            jax.ShapeDtypeStruct((_N,), jnp.float32),
        ),
    )(hist, x, w)


def kernel(X, W, b, cluster_assignment, full_edge_index):
    n, d = X.shape
    hist, csrc, cdst, cnt = _deg_call(full_edge_index, cluster_assignment)
    y2, dinv = _dy_call(hist, X, W)
    return _agg_call(y2,
                     csrc.reshape(_NW, _CREG, _CWA),
                     cdst.reshape(_NW, _CREG, _CWA),
                     cnt.reshape(_NS, 2, _L),
                     dinv, b)
